# Initial kernel scaffold; baseline (speedup 1.0000x reference)
import jax
import jax.numpy as jnp
from jax.experimental import pallas as pl


def kernel(xyz, points, W0, W1, W2):
    raise NotImplementedError("write your pallas kernel here")


# speedup vs baseline: 12.3287x; 12.3287x over previous
import functools

import jax
import jax.numpy as jnp
from jax import lax
from jax.experimental import pallas as pl
from jax.experimental.pallas import tpu as pltpu, tpu_sc as plsc

S = 1024
K = 32
QT = 128
QT2 = 256
_HI = 3.0e38


def _leaky(x):
    return jnp.where(x >= 0, x, 0.1 * x)


def _dot(a, b):
    return lax.dot_general(a, b, (((1,), (0,)), ((), ())))


def _proj_body(ptsT_ref, xyzT_ref, w0pT_ref, f_ref):
    p = ptsT_ref[0]
    xt = xyzT_ref[0]
    fp = _dot(p, w0pT_ref[...])
    n = fp.shape[0]
    f_ref[0] = jnp.concatenate(
        [fp, xt, jnp.zeros((n, 61), jnp.float32)], axis=1)


def _proj(pointsT, xyzT, W0pT):
    B, N, _ = pointsT.shape
    NT = 2048
    return pl.pallas_call(
        _proj_body,
        grid=(B, N // NT),
        in_specs=[
            pl.BlockSpec((1, NT, 64), lambda b, t: (b, t, 0)),
            pl.BlockSpec((1, NT, 3), lambda b, t: (b, t, 0)),
            pl.BlockSpec((64, 64), lambda b, t: (0, 0)),
        ],
        out_specs=pl.BlockSpec((1, NT, 128), lambda b, t: (b, t, 0)),
        out_shape=jax.ShapeDtypeStruct((B, N, 128), jnp.float32),
    )(pointsT, xyzT, W0pT)


def _fps_body(xyz_ref, idx_ref, nxyz_ref):
    B = xyz_ref.shape[0]
    R, C = xyz_ref.shape[2], xyz_ref.shape[3]
    x = xyz_ref[:, 0]
    y = xyz_ref[:, 1]
    z = xyz_ref[:, 2]
    li = (lax.broadcasted_iota(jnp.int32, (B, R, C), 1) * C
          + lax.broadcasted_iota(jnp.int32, (B, R, C), 2))
    tj = lax.broadcasted_iota(jnp.int32, (B, S), 1)

    def body(t, carry):
        dists, far, idxs, nx, ny, nz = carry
        onehot = li == far
        cx = jnp.sum(jnp.sum(jnp.where(onehot, x, 0.0), axis=2, keepdims=True),
                     axis=1, keepdims=True)
        cy = jnp.sum(jnp.sum(jnp.where(onehot, y, 0.0), axis=2, keepdims=True),
                     axis=1, keepdims=True)
        cz = jnp.sum(jnp.sum(jnp.where(onehot, z, 0.0), axis=2, keepdims=True),
                     axis=1, keepdims=True)
        dx = x - cx
        dy = y - cy
        dz = z - cz
        d = (dx * dx + dy * dy) + dz * dz
        dists = jnp.minimum(dists, d)
        m = jnp.max(jnp.max(dists, axis=2, keepdims=True), axis=1,
                    keepdims=True)
        nxt = jnp.min(jnp.min(jnp.where(dists == m, li, jnp.int32(R * C)),
                              axis=2, keepdims=True), axis=1, keepdims=True)
        sel = tj == t
        idxs = jnp.where(sel, jnp.reshape(far, (B, 1)), idxs)
        nx = jnp.where(sel, jnp.reshape(cx, (B, 1)), nx)
        ny = jnp.where(sel, jnp.reshape(cy, (B, 1)), ny)
        nz = jnp.where(sel, jnp.reshape(cz, (B, 1)), nz)
        return dists, nxt, idxs, nx, ny, nz

    init = (jnp.full((B, R, C), 1e10, jnp.float32),
            jnp.zeros((B, 1, 1), jnp.int32),
            jnp.zeros((B, S), jnp.int32),
            jnp.zeros((B, S), jnp.float32),
            jnp.zeros((B, S), jnp.float32),
            jnp.zeros((B, S), jnp.float32))
    _, _, idxs, nx, ny, nz = lax.fori_loop(0, S, body, init)
    idx_ref[...] = idxs
    nxyz_ref[:, 0, :] = nx
    nxyz_ref[:, 1, :] = ny
    nxyz_ref[:, 2, :] = nz


def _fps(xyz):
    B, _, N = xyz.shape
    xyz4 = xyz.reshape(B, 3, 8, N // 8)
    return pl.pallas_call(
        _fps_body,
        out_shape=(jax.ShapeDtypeStruct((B, S), jnp.int32),
                   jax.ShapeDtypeStruct((B, 3, S), jnp.float32)),
    )(xyz4)


def _topk_body(q_ref, xyz_ref, knn_ref, *, N):
    b = pl.program_id(0)
    q = q_ref[0]
    x = xyz_ref[0]
    qdot = _dot(q, x)
    qn = jnp.sum(q * q, axis=1, keepdims=True)
    xn = jnp.sum(x * x, axis=0, keepdims=True)
    sqrd = (-2.0 * qdot + qn) + xn
    ji = lax.broadcasted_iota(jnp.int32, (QT, N), 1)
    base = b * N
    cols = []
    for _ in range(K):
        m = jnp.min(sqrd, axis=1, keepdims=True)
        idx = jnp.min(jnp.where(sqrd == m, ji, jnp.int32(N)), axis=1,
                      keepdims=True)
        cols.append(idx + base)
        sqrd = jnp.where(ji == idx, _HI, sqrd)
    knn_ref[0] = jnp.concatenate(cols, axis=1)


def _topk(new_xyz_t, xyz):
    B, _, N = xyz.shape
    return pl.pallas_call(
        functools.partial(_topk_body, N=N),
        grid=(B, S // QT),
        in_specs=[
            pl.BlockSpec((1, QT, 3), lambda b, q: (b, q, 0)),
            pl.BlockSpec((1, 3, N), lambda b, q: (b, 0, 0)),
        ],
        out_specs=pl.BlockSpec((1, QT, K), lambda b, q: (b, q, 0)),
        out_shape=jax.ShapeDtypeStruct((B, S, K), jnp.int32),
    )(new_xyz_t, xyz)


def _sc_gather(f_flat, idx3):
    info = plsc.get_sparse_core_info()
    NC, NS = info.num_cores, info.num_subcores
    NW = NC * NS
    CH = idx3.shape[1]
    per_w = CH * 128
    GROUP = 4
    rows_per_group = GROUP * 128
    mesh = plsc.VectorSubcoreMesh(core_axis_name="c", subcore_axis_name="s")

    @functools.partial(
        pl.kernel, mesh=mesh,
        out_type=jax.ShapeDtypeStruct((NW * per_w, 128), jnp.float32),
        scratch_types=[
            pltpu.VMEM((CH, 128), jnp.int32),
            pltpu.VMEM((rows_per_group, 128), jnp.float32),
            pltpu.SemaphoreType.DMA,
        ],
    )
    def gather_k(f_hbm, idx_hbm, out_hbm, idx_v, rows_v, sem):
        wid = lax.axis_index("s") * NC + lax.axis_index("c")
        pltpu.sync_copy(idx_hbm.at[wid], idx_v)
        base = wid * per_w
        for g in range(CH // GROUP):
            handles = []
            for c in range(GROUP):
                h = pltpu.async_copy(
                    f_hbm.at[idx_v.at[g * GROUP + c]],
                    rows_v.at[pl.ds(c * 128, 128)], sem)
                handles.append(h)
            for h in handles:
                h.wait()
            pltpu.sync_copy(rows_v,
                            out_hbm.at[pl.ds(base + g * rows_per_group,
                                             rows_per_group)])

    return gather_k(f_flat, idx3)


def _mlp_body(gfp_ref, gfx_ref, q_ref, w0xTp_ref, w1T_ref, w2T_ref, out_ref):
    pts_g = gfp_ref[0]
    gx = gfx_ref[0]
    qpad = q_ref[0]
    gxn = gx - qpad
    h0 = _leaky(_dot(gxn, w0xTp_ref[...]) + pts_g)
    h1 = _leaky(_dot(h0, w1T_ref[...]))
    h2 = _leaky(_dot(h1, w2T_ref[...]))
    h3 = h2.reshape(QT2, K, 128)
    acc = h3[:, 0, :]
    for k in range(1, K):
        acc = jnp.maximum(acc, h3[:, k, :])
    out_ref[0] = acc


def _mlp(gfp, gfx, q_rep, W0xTp, W1T, W2T):
    B = q_rep.shape[0]
    return pl.pallas_call(
        _mlp_body,
        grid=(B, S // QT2),
        in_specs=[
            pl.BlockSpec((1, QT2 * K, 64), lambda b, q: (b, q, 0)),
            pl.BlockSpec((1, QT2 * K, 64), lambda b, q: (b, q, 0)),
            pl.BlockSpec((1, QT2 * K, 64), lambda b, q: (b, q, 0)),
            pl.BlockSpec((64, 64), lambda b, q: (0, 0)),
            pl.BlockSpec((64, 64), lambda b, q: (0, 0)),
            pl.BlockSpec((64, 128), lambda b, q: (0, 0)),
        ],
        out_specs=pl.BlockSpec((1, QT2, 128), lambda b, q: (b, q, 0)),
        out_shape=jax.ShapeDtypeStruct((B, S, 128), jnp.float32),
    )(gfp, gfx, q_rep, W0xTp, W1T, W2T)


def kernel(xyz, points, W0, W1, W2):
    B, _, N = xyz.shape
    pointsT = jnp.transpose(points, (0, 2, 1))
    xyzT = jnp.transpose(xyz, (0, 2, 1))
    W0pT = jnp.transpose(W0[:, 3:], (1, 0))
    W0xTp = jnp.concatenate(
        [jnp.transpose(W0[:, 0:3], (1, 0)),
         jnp.zeros((61, 64), jnp.float32)], axis=0)
    W1T = jnp.transpose(W1, (1, 0))
    W2T = jnp.transpose(W2, (1, 0))
    F = _proj(pointsT, xyzT, W0pT)
    fps_idx, new_xyz = _fps(xyz)
    new_xyz_t = jnp.transpose(new_xyz, (0, 2, 1))
    knn = _topk(new_xyz_t, xyz)
    idx3 = knn.reshape(32, (B * S * K) // (32 * 128), 128)
    GF = _sc_gather(F.reshape(B * N, 128), idx3)
    GF4 = GF.reshape(B, S * K, 128)
    q_rep = jnp.concatenate(
        [jnp.repeat(new_xyz_t, K, axis=1),
         jnp.zeros((B, S * K, 61), jnp.float32)], axis=2)
    NP = _mlp(GF4[..., 0:64], GF4[..., 64:128], q_rep, W0xTp, W1T, W2T)
    return new_xyz, jnp.transpose(NP, (0, 2, 1)), fps_idx
